# Initial kernel scaffold; baseline (speedup 1.0000x reference)
import jax
import jax.numpy as jnp
from jax.experimental import pallas as pl


def kernel(t, pos_embedding):
    raise NotImplementedError("write your pallas kernel here")


# speedup vs baseline: 6.1818x; 6.1818x over previous
import functools

import jax
import jax.numpy as jnp
from jax import lax
from jax.experimental import pallas as pl
from jax.experimental.pallas import tpu as pltpu
from jax.experimental.pallas import tpu_sc as plsc

EMBED = 128
G = 128
K = 8
BLK = G * K


def _sc_gather(idx2d, table):
    n_rows, g = idx2d.shape
    B = n_rows * g
    info = plsc.get_sparse_core_info()
    nw = info.num_cores * info.num_subcores
    b_per_w = B // nw
    n_blocks = b_per_w // BLK
    mesh = plsc.VectorSubcoreMesh(core_axis_name="c", subcore_axis_name="s")

    @functools.partial(
        pl.kernel,
        mesh=mesh,
        out_type=jax.ShapeDtypeStruct((B, EMBED), jnp.float32),
        scratch_types=[
            pltpu.VMEM((K, G), jnp.int32),
            pltpu.VMEM((G, EMBED), jnp.float32),
            pltpu.SemaphoreType.DMA,
        ],
    )
    def k(idx_hbm, table_hbm, out_hbm, idx_v, rows_v, sem):
        wid = lax.axis_index("s") * info.num_cores + lax.axis_index("c")
        base = wid * b_per_w

        def body(i, carry):
            blk_base = pl.multiple_of(base + i * BLK, BLK)
            pltpu.sync_copy(idx_hbm.at[pl.ds(pl.multiple_of(blk_base // G, K), K)], idx_v)
            for j in range(K):
                pltpu.async_copy(table_hbm.at[idx_v.at[j]], rows_v, sem).wait()
                pltpu.sync_copy(rows_v, out_hbm.at[pl.ds(pl.multiple_of(blk_base + j * G, G), G)])
            return carry

        lax.fori_loop(0, n_blocks, body, 0)

    return k(idx2d, table)


def kernel(t, pos_embedding):
    b, h = t.shape
    idx2d = t.astype(jnp.int32).reshape(b * h // G, G)
    out = _sc_gather(idx2d, pos_embedding)
    return out.reshape(b, h, EMBED)


# speedup vs baseline: 6.8983x; 1.1159x over previous
import functools

import jax
import jax.numpy as jnp
from jax import lax
from jax.experimental import pallas as pl
from jax.experimental.pallas import tpu as pltpu
from jax.experimental.pallas import tpu_sc as plsc

EMBED = 128
G = 128
C = 256
U = 8
BLK = U * C


def _sc_gather(idx2d, table):
    n_rows, g = idx2d.shape
    B = n_rows * g
    info = plsc.get_sparse_core_info()
    nw = info.num_cores * info.num_subcores
    b_per_w = B // nw
    n_blocks = b_per_w // BLK
    mesh = plsc.VectorSubcoreMesh(core_axis_name="c", subcore_axis_name="s")

    @functools.partial(
        pl.kernel,
        mesh=mesh,
        out_type=jax.ShapeDtypeStruct((B, EMBED), jnp.float32),
        scratch_types=[
            pltpu.VMEM((BLK // G, G), jnp.int32),
            pltpu.VMEM((C, EMBED), jnp.float32),
            pltpu.VMEM((C, EMBED), jnp.float32),
            pltpu.SemaphoreType.DMA,
            pltpu.SemaphoreType.DMA,
            pltpu.SemaphoreType.DMA,
            pltpu.SemaphoreType.DMA,
        ],
    )
    def k(idx_hbm, table_hbm, out_hbm, idx_v, rows0, rows1, g0, g1, s0, s1):
        wid = lax.axis_index("s") * info.num_cores + lax.axis_index("c")
        base = wid * b_per_w
        rows = (rows0, rows1)
        gsem = (g0, g1)
        ssem = (s0, s1)

        def fire_gathers(u, blk):
            del blk
            buf = rows[u % 2]
            return [
                pltpu.async_copy(
                    table_hbm.at[idx_v.at[u * (C // G) + h]],
                    buf.at[pl.ds(h * G, G)],
                    gsem[u % 2],
                )
                for h in range(C // G)
            ]

        def fire_store(u, blk):
            dst = out_hbm.at[pl.ds(pl.multiple_of(blk + u * C, C), C)]
            return pltpu.async_copy(rows[u % 2], dst, ssem[u % 2])

        def body(i, carry):
            blk = pl.multiple_of(base + i * BLK, BLK)
            pltpu.sync_copy(
                idx_hbm.at[pl.ds(pl.multiple_of(blk // G, BLK // G), BLK // G)],
                idx_v,
            )
            gh = {0: fire_gathers(0, blk)}
            sh = {}
            for u in range(1, U):
                for h in gh[u - 1]:
                    h.wait()
                sh[u - 1] = fire_store(u - 1, blk)
                if u >= 2:
                    sh[u - 2].wait()
                gh[u] = fire_gathers(u, blk)
            for h in gh[U - 1]:
                h.wait()
            sh[U - 1] = fire_store(U - 1, blk)
            sh[U - 2].wait()
            sh[U - 1].wait()
            return carry

        lax.fori_loop(0, n_blocks, body, 0)

    return k(idx2d, table)


def kernel(t, pos_embedding):
    b, h = t.shape
    idx2d = t.astype(jnp.int32).reshape(b * h // G, G)
    out = _sc_gather(idx2d, pos_embedding)
    return out.reshape(b, h, EMBED)


# speedup vs baseline: 17.2768x; 2.5045x over previous
import functools

import jax
import jax.numpy as jnp
from jax import lax
from jax.experimental import pallas as pl
from jax.experimental.pallas import tpu as pltpu
from jax.experimental.pallas import tpu_sc as plsc

EMBED = 128
G = 128
C = 256
U = 8
BLK = U * C


def _sc_gather(idx2d, table):
    n_rows, g = idx2d.shape
    B = n_rows * g
    info = plsc.get_sparse_core_info()
    nw = info.num_cores * info.num_subcores
    b_per_w = B // nw
    n_blocks = b_per_w // BLK
    mesh = plsc.VectorSubcoreMesh(core_axis_name="c", subcore_axis_name="s")

    @functools.partial(
        pl.kernel,
        mesh=mesh,
        out_type=jax.ShapeDtypeStruct((B, EMBED), jnp.float32),
        scratch_types=[
            pltpu.VMEM((BLK // G, G), jnp.int32),
            pltpu.VMEM((C, EMBED), jnp.float32),
            pltpu.VMEM((C, EMBED), jnp.float32),
            pltpu.VMEM_SHARED((1001, EMBED), jnp.float32),
            pltpu.SemaphoreType.DMA,
            pltpu.SemaphoreType.DMA,
            pltpu.SemaphoreType.DMA,
            pltpu.SemaphoreType.DMA,
        ],
    )
    def k(idx_hbm, table_hbm, out_hbm, idx_v, rows0, rows1, table_sh, g0, g1, s0, s1):
        sid = lax.axis_index("s")
        wid = sid * info.num_cores + lax.axis_index("c")
        base = wid * b_per_w
        rows = (rows0, rows1)
        gsem = (g0, g1)
        ssem = (s0, s1)

        @pl.when(sid == 0)
        def _():
            pltpu.sync_copy(table_hbm, table_sh)

        plsc.subcore_barrier()

        def fire_gathers(u, blk):
            del blk
            buf = rows[u % 2]
            return [
                pltpu.async_copy(
                    table_sh.at[idx_v.at[u * (C // G) + h]],
                    buf.at[pl.ds(h * G, G)],
                    gsem[u % 2],
                )
                for h in range(C // G)
            ]

        def fire_store(u, blk):
            dst = out_hbm.at[pl.ds(pl.multiple_of(blk + u * C, C), C)]
            return pltpu.async_copy(rows[u % 2], dst, ssem[u % 2])

        def body(i, carry):
            blk = pl.multiple_of(base + i * BLK, BLK)
            pltpu.sync_copy(
                idx_hbm.at[pl.ds(pl.multiple_of(blk // G, BLK // G), BLK // G)],
                idx_v,
            )
            gh = {0: fire_gathers(0, blk)}
            sh = {}
            for u in range(1, U):
                for h in gh[u - 1]:
                    h.wait()
                sh[u - 1] = fire_store(u - 1, blk)
                if u >= 2:
                    sh[u - 2].wait()
                gh[u] = fire_gathers(u, blk)
            for h in gh[U - 1]:
                h.wait()
            sh[U - 1] = fire_store(U - 1, blk)
            sh[U - 2].wait()
            sh[U - 1].wait()
            return carry

        lax.fori_loop(0, n_blocks, body, 0)

    return k(idx2d, table)


def kernel(t, pos_embedding):
    b, h = t.shape
    idx2d = t.astype(jnp.int32).reshape(b * h // G, G)
    out = _sc_gather(idx2d, pos_embedding)
    return out.reshape(b, h, EMBED)


# speedup vs baseline: 18.1161x; 1.0486x over previous
import functools

import jax
import jax.numpy as jnp
from jax import lax
from jax.experimental import pallas as pl
from jax.experimental.pallas import tpu as pltpu
from jax.experimental.pallas import tpu_sc as plsc

EMBED = 128
G = 128
C = 256
U = 8
BLK = U * C


def _sc_gather(idx2d, table):
    n_rows, g = idx2d.shape
    B = n_rows * g
    info = plsc.get_sparse_core_info()
    nw = info.num_cores * info.num_subcores
    b_per_w = B // nw
    n_blocks = b_per_w // BLK
    mesh = plsc.VectorSubcoreMesh(core_axis_name="c", subcore_axis_name="s")

    @functools.partial(
        pl.kernel,
        mesh=mesh,
        out_type=jax.ShapeDtypeStruct((B, EMBED), jnp.float32),
        scratch_types=[
            pltpu.VMEM((BLK // G, G), jnp.int32),
            pltpu.VMEM((C, EMBED), jnp.float32),
            pltpu.VMEM((C, EMBED), jnp.float32),
            pltpu.VMEM_SHARED((1001, EMBED), jnp.float32),
            pltpu.SemaphoreType.DMA,
            pltpu.SemaphoreType.DMA,
            pltpu.SemaphoreType.DMA,
            pltpu.SemaphoreType.DMA,
        ],
    )
    def k(idx_hbm, table_hbm, out_hbm, idx_v, rows0, rows1, table_sh, g0, g1, s0, s1):
        sid = lax.axis_index("s")
        wid = sid * info.num_cores + lax.axis_index("c")
        base = wid * b_per_w
        rows = (rows0, rows1)
        gsem = (g0, g1)
        ssem = (s0, s1)

        @pl.when(sid == 0)
        def _():
            pltpu.sync_copy(table_hbm, table_sh)

        plsc.subcore_barrier()

        def fire_gathers(u, blk):
            del blk
            buf = rows[u % 2]
            return [
                pltpu.async_copy(
                    table_sh.at[idx_v.at[u * (C // G) + h]],
                    buf.at[pl.ds(h * G, G)],
                    gsem[u % 2],
                )
                for h in range(C // G)
            ]

        def fire_store(u, blk):
            dst = out_hbm.at[pl.ds(pl.multiple_of(blk + u * C, C), C)]
            return pltpu.async_copy(rows[u % 2], dst, ssem[u % 2])

        def store_wait(par):
            pltpu.make_async_copy(
                rows[par], out_hbm.at[pl.ds(base, C)], ssem[par]
            ).wait()

        def body(i, carry):
            blk = pl.multiple_of(base + i * BLK, BLK)
            pltpu.sync_copy(
                idx_hbm.at[pl.ds(pl.multiple_of(blk // G, BLK // G), BLK // G)],
                idx_v,
            )
            gh = {}
            sh = {}
            for u in range(U):
                if u >= 2:
                    sh[u - 2].wait()
                else:
                    @pl.when(i > 0)
                    def _(par=u % 2):
                        store_wait(par)

                gh[u] = fire_gathers(u, blk)
                if u >= 1:
                    for h in gh[u - 1]:
                        h.wait()
                    sh[u - 1] = fire_store(u - 1, blk)
            for h in gh[U - 1]:
                h.wait()
            sh[U - 1] = fire_store(U - 1, blk)
            return carry

        lax.fori_loop(0, n_blocks, body, 0)
        store_wait((U - 2) % 2)
        store_wait((U - 1) % 2)

    return k(idx2d, table)


def kernel(t, pos_embedding):
    b, h = t.shape
    idx2d = t.astype(jnp.int32).reshape(b * h // G, G)
    out = _sc_gather(idx2d, pos_embedding)
    return out.reshape(b, h, EMBED)


# speedup vs baseline: 19.1203x; 1.0554x over previous
import functools

import jax
import jax.numpy as jnp
from jax import lax
from jax.experimental import pallas as pl
from jax.experimental.pallas import tpu as pltpu
from jax.experimental.pallas import tpu_sc as plsc

EMBED = 128
G = 128
C = 256
U = 4
BLK = U * C
SUPER = 2 * BLK


def _sc_gather(idx2d, table):
    n_rows, g = idx2d.shape
    B = n_rows * g
    info = plsc.get_sparse_core_info()
    nw = info.num_cores * info.num_subcores
    b_per_w = B // nw
    n_super = b_per_w // SUPER
    mesh = plsc.VectorSubcoreMesh(core_axis_name="c", subcore_axis_name="s")

    @functools.partial(
        pl.kernel,
        mesh=mesh,
        out_type=jax.ShapeDtypeStruct((B, EMBED), jnp.float32),
        scratch_types=[
            pltpu.VMEM((BLK // G, G), jnp.int32),
            pltpu.VMEM((BLK // G, G), jnp.int32),
            pltpu.VMEM((C, EMBED), jnp.float32),
            pltpu.VMEM((C, EMBED), jnp.float32),
            pltpu.VMEM_SHARED((1001, EMBED), jnp.float32),
            pltpu.SemaphoreType.DMA,
            pltpu.SemaphoreType.DMA,
            pltpu.SemaphoreType.DMA,
            pltpu.SemaphoreType.DMA,
            pltpu.SemaphoreType.DMA,
            pltpu.SemaphoreType.DMA,
        ],
    )
    def k(idx_hbm, table_hbm, out_hbm, idx0, idx1, rows0, rows1, table_sh,
          g0, g1, s0, s1, i0, i1):
        sid = lax.axis_index("s")
        wid = sid * info.num_cores + lax.axis_index("c")
        base = wid * b_per_w
        rows = (rows0, rows1)
        idxs = (idx0, idx1)
        gsem = (g0, g1)
        ssem = (s0, s1)
        isem = (i0, i1)

        @pl.when(sid == 0)
        def _():
            pltpu.sync_copy(table_hbm, table_sh)

        plsc.subcore_barrier()

        def idx_src(block):
            off = pl.multiple_of(block * (BLK // G), BLK // G)
            return idx_hbm.at[pl.ds(off, BLK // G)]

        def fire_idx(block, par):
            return pltpu.async_copy(idx_src(block), idxs[par], isem[par])

        def idx_wait(par):
            pltpu.make_async_copy(idx_src(0), idxs[par], isem[par]).wait()

        def fire_gathers(u, ipar):
            buf = rows[u % 2]
            return [
                pltpu.async_copy(
                    table_sh.at[idxs[ipar].at[(u % U) * (C // G) + h]],
                    buf.at[pl.ds(h * G, G)],
                    gsem[u % 2],
                )
                for h in range(C // G)
            ]

        def fire_store(u, blk):
            dst = out_hbm.at[pl.ds(pl.multiple_of(blk + u * C, C), C)]
            return pltpu.async_copy(rows[u % 2], dst, ssem[u % 2])

        def store_wait(par):
            pltpu.make_async_copy(
                rows[par], out_hbm.at[pl.ds(base, C)], ssem[par]
            ).wait()

        wblock0 = wid * (b_per_w // BLK)
        fire_idx(wblock0, 0)
        fire_idx(wblock0 + 1, 1)

        def body(i, carry):
            blk = pl.multiple_of(base + i * SUPER, SUPER)
            gh = {}
            sh = {}
            for u in range(2 * U):
                ipar = u // U
                if u == 0:
                    idx_wait(0)
                if u == U:
                    idx_wait(1)
                if u >= 2:
                    sh[u - 2].wait()
                else:
                    @pl.when(i > 0)
                    def _(par=u % 2):
                        store_wait(par)

                gh[u] = fire_gathers(u, ipar)
                if u >= 1:
                    for h in gh[u - 1]:
                        h.wait()
                    sh[u - 1] = fire_store(u - 1, blk)
                if u == U:
                    @pl.when(i + 1 < n_super)
                    def _():
                        fire_idx(wblock0 + 2 * (i + 1), 0)
            for h in gh[2 * U - 1]:
                h.wait()
            sh[2 * U - 1] = fire_store(2 * U - 1, blk)

            @pl.when(i + 1 < n_super)
            def _():
                fire_idx(wblock0 + 2 * (i + 1) + 1, 1)

            return carry

        lax.fori_loop(0, n_super, body, 0)
        store_wait(0)
        store_wait(1)

    return k(idx2d, table)


def kernel(t, pos_embedding):
    b, h = t.shape
    idx2d = t.astype(jnp.int32).reshape(b * h // G, G)
    out = _sc_gather(idx2d, pos_embedding)
    return out.reshape(b, h, EMBED)
